# baseline (speedup 1.0000x reference)
import jax
import jax.numpy as jnp
from jax import lax
from jax.experimental import pallas as pl
from jax.experimental.pallas import tpu as pltpu
from jax.experimental.pallas import tpu_sc as plsc

B = 256
NCH = 50000
NN = 10000
FAN = 16

NC = 2
NS = 16
NW = NC * NS
NPW = 320
N_PAD = NW * NPW
GSZ = 8
GROUPS = NPW // GSZ

NCH_PAD = 50176
W_A = 3584
R_C = 1280


def _exp_t_body(x_ref, o_ref):
    o_ref[...] = jnp.exp(x_ref[...]).T


def _log_t_body(s_ref, o_ref):
    o_ref[...] = jnp.log(s_ref[...]).T


def _sc_body(et_hbm, idx_hbm, lwb_hbm, out_hbm, idx_v, lwb_v, rows_v, out_v, sem):
    wid = lax.axis_index("s") * NC + lax.axis_index("c")
    pltpu.sync_copy(idx_hbm.at[wid], idx_v)
    base = wid * NPW

    def group(g, carry):
        pltpu.async_copy(et_hbm.at[idx_v.at[g]], rows_v, sem).wait()
        pltpu.sync_copy(lwb_hbm.at[wid * GROUPS + g], lwb_v)
        for k in range(GSZ):
            wv = [jnp.exp(lwb_v[k * FAN + r]) for r in range(FAN)]

            def chunk(c, _, k=k, wv=wv):
                acc = jnp.zeros((16,), jnp.float32)
                for r in range(FAN):
                    acc = acc + wv[r] * rows_v[k * FAN + r, pl.ds(c * 16, 16)]
                out_v[k, pl.ds(c * 16, 16)] = acc
                return 0

            lax.fori_loop(0, B // 16, chunk, 0)
        pltpu.sync_copy(out_v, out_hbm.at[pl.ds(base + g * GSZ, GSZ)])
        return carry

    lax.fori_loop(0, GROUPS, group, 0)


def _sc_call(et, idx3, lwb3):
    mesh = plsc.VectorSubcoreMesh(core_axis_name="c", subcore_axis_name="s")
    f = pl.kernel(
        _sc_body,
        out_type=jax.ShapeDtypeStruct((N_PAD, B), jnp.float32),
        mesh=mesh,
        scratch_types=[
            pltpu.VMEM((GROUPS, GSZ * FAN), jnp.int32),
            pltpu.VMEM((GSZ * FAN, FAN), jnp.float32),
            pltpu.VMEM((GSZ * FAN, B), jnp.float32),
            pltpu.VMEM((GSZ, B), jnp.float32),
            pltpu.SemaphoreType.DMA,
        ],
    )
    return f(et, idx3, lwb3)


def kernel(x, idxs, log_weights):
    x_p = jnp.pad(x, ((0, 0), (0, NCH_PAD - NCH)))
    et = pl.pallas_call(
        _exp_t_body,
        grid=(NCH_PAD // W_A,),
        in_specs=[pl.BlockSpec((B, W_A), lambda i: (0, i))],
        out_specs=pl.BlockSpec((W_A, B), lambda i: (i, 0)),
        out_shape=jax.ShapeDtypeStruct((NCH_PAD, B), jnp.float32),
    )(x_p)

    idx_p = jnp.pad(idxs, ((0, N_PAD - NN), (0, 0)))
    lw_p = jnp.pad(log_weights, ((0, N_PAD - NN), (0, 0)))
    idx3 = idx_p.reshape(NW, GROUPS, GSZ * FAN)
    lwb3 = jnp.broadcast_to(
        lw_p.reshape(N_PAD * FAN, 1), (N_PAD * FAN, 16)
    ).reshape(NW * GROUPS, GSZ * FAN, 16)

    s_pad = _sc_call(et, idx3, lwb3)

    lls_pad = pl.pallas_call(
        _log_t_body,
        grid=(N_PAD // R_C,),
        in_specs=[pl.BlockSpec((R_C, B), lambda i: (i, 0))],
        out_specs=pl.BlockSpec((B, R_C), lambda i: (0, i)),
        out_shape=jax.ShapeDtypeStruct((B, N_PAD), jnp.float32),
    )(s_pad)
    return lls_pad[:, :NN]


# speedup vs baseline: 1.7384x; 1.7384x over previous
import jax
import jax.numpy as jnp
from jax import lax
from jax.experimental import pallas as pl
from jax.experimental.pallas import tpu as pltpu
from jax.experimental.pallas import tpu_sc as plsc

B = 256
NCH = 50000
NN = 10000
FAN = 16

NC = 2
NS = 16
NW = NC * NS
NPW = 320
N_PAD = NW * NPW
GSZ = 8
GROUPS = NPW // GSZ

R_C = 1280

_BCAST_DNUMS = lax.GatherDimensionNumbers(
    offset_dims=(), collapsed_slice_dims=(0,), start_index_map=(0,))


def _bcast_lane(v, r):
    idx = jnp.full((16, 1), r, jnp.int32)
    return lax.gather(v, idx, _BCAST_DNUMS, (1,),
                      mode=lax.GatherScatterMode.PROMISE_IN_BOUNDS)


def _log_t_body(s_ref, o_ref):
    o_ref[...] = jnp.log(s_ref[...]).T


def _sc_body(et_hbm, idx_hbm, lw_hbm, out_hbm,
             idx_v, lw_v, rows_v, out_v, gsem0, gsem1, osem0, osem1):
    wid = lax.axis_index("s") * NC + lax.axis_index("c")
    pltpu.sync_copy(idx_hbm.at[wid], idx_v)
    pltpu.sync_copy(lw_hbm.at[wid], lw_v)
    base = wid * NPW
    gsems = (gsem0, gsem1)
    osems = (osem0, osem1)

    pltpu.async_copy(et_hbm.at[idx_v.at[0]], rows_v.at[0], gsems[0])

    def pair(t, carry):
        for b in range(2):
            g = t * 2 + b
            nb = 1 - b

            @pl.when(g + 1 < GROUPS)
            def _prefetch(g=g, nb=nb):
                pltpu.async_copy(et_hbm.at[idx_v.at[g + 1]],
                                 rows_v.at[nb], gsems[nb])

            pltpu.make_async_copy(et_hbm.at[pl.ds(0, GSZ * FAN)],
                                  rows_v.at[b], gsems[b]).wait()

            @pl.when(t >= 1)
            def _reclaim(b=b):
                pltpu.make_async_copy(out_v.at[b],
                                      out_hbm.at[pl.ds(0, GSZ)],
                                      osems[b]).wait()

            for k in range(GSZ):
                lwv = lw_v[g, pl.ds(k * FAN, FAN)]
                lwb = [_bcast_lane(lwv, r) for r in range(FAN)]

                def chunk(c, _, k=k, b=b, lwb=lwb):
                    acc = jnp.zeros((16,), jnp.float32)
                    for r in range(FAN):
                        acc = acc + jnp.exp(
                            lwb[r] + rows_v[b, k * FAN + r,
                                            pl.ds(c * 16, 16)])
                    out_v[b, k, pl.ds(c * 16, 16)] = acc
                    return 0

                lax.fori_loop(0, B // 16, chunk, 0)

            pltpu.async_copy(out_v.at[b],
                             out_hbm.at[pl.ds(base + g * GSZ, GSZ)],
                             osems[b])
        return carry

    lax.fori_loop(0, GROUPS // 2, pair, 0)

    pltpu.make_async_copy(out_v.at[0], out_hbm.at[pl.ds(0, GSZ)], osems[0]).wait()
    pltpu.make_async_copy(out_v.at[1], out_hbm.at[pl.ds(0, GSZ)], osems[1]).wait()


def _sc_call(et, idx3, lw3):
    mesh = plsc.VectorSubcoreMesh(core_axis_name="c", subcore_axis_name="s")
    f = pl.kernel(
        _sc_body,
        out_type=jax.ShapeDtypeStruct((N_PAD, B), jnp.float32),
        mesh=mesh,
        scratch_types=[
            pltpu.VMEM((GROUPS, GSZ * FAN), jnp.int32),
            pltpu.VMEM((GROUPS, GSZ * FAN), jnp.float32),
            pltpu.VMEM((2, GSZ * FAN, B), jnp.float32),
            pltpu.VMEM((2, GSZ, B), jnp.float32),
            pltpu.SemaphoreType.DMA,
            pltpu.SemaphoreType.DMA,
            pltpu.SemaphoreType.DMA,
            pltpu.SemaphoreType.DMA,
        ],
    )
    return f(et, idx3, lw3)


def kernel(x, idxs, log_weights):
    xt = x.T

    idx_p = jnp.pad(idxs, ((0, N_PAD - NN), (0, 0)))
    lw_p = jnp.pad(log_weights, ((0, N_PAD - NN), (0, 0)))
    idx3 = idx_p.reshape(NW, GROUPS, GSZ * FAN)
    lw3 = lw_p.reshape(NW, GROUPS, GSZ * FAN)

    s_pad = _sc_call(xt, idx3, lw3)

    lls_pad = pl.pallas_call(
        _log_t_body,
        grid=(N_PAD // R_C,),
        in_specs=[pl.BlockSpec((R_C, B), lambda i: (i, 0))],
        out_specs=pl.BlockSpec((B, R_C), lambda i: (0, i)),
        out_shape=jax.ShapeDtypeStruct((B, N_PAD), jnp.float32),
    )(s_pad)
    return lls_pad[:, :NN]
